# baseline (speedup 1.0000x reference)
import functools

import jax
import jax.numpy as jnp
from jax import lax
from jax.experimental import pallas as pl
from jax.experimental.pallas import tpu as pltpu
from jax.experimental.pallas import tpu_sc as plsc

N = 10000
D = 128
E = 320000

NC = 2
NS = 16
NW = NC * NS

CHUNK = 128
NBUF = 2
CPT = 80
PER_TILE = CPT * CHUNK
EPAD = PER_TILE * NW

NPAD = 10112
RPT = NPAD // NS

_MESH = plsc.VectorSubcoreMesh(core_axis_name="c", subcore_axis_name="s")


@functools.partial(
    pl.kernel,
    out_type=jax.ShapeDtypeStruct((NC, NPAD, D), jnp.float32),
    mesh=_MESH,
    scratch_types=[
        [pltpu.VMEM((CHUNK,), jnp.int32) for _ in range(NBUF)],
        [pltpu.VMEM((CHUNK,), jnp.int32) for _ in range(NBUF)],
        [pltpu.VMEM((CHUNK, D), jnp.float32) for _ in range(NBUF)],
        pltpu.VMEM_SHARED((NPAD, D), jnp.float32),
        [pltpu.SemaphoreType.DMA for _ in range(NBUF)],
    ],
)
def _sc_aggregate(x_hbm, src_hbm, dst_hbm, zeros_hbm, out_hbm,
                  src_i, dst_i, rows, acc_sh, gsems):
    c = lax.axis_index("c")
    s = lax.axis_index("s")
    row0 = s * RPT
    base = (c * NS + s) * PER_TILE

    pltpu.sync_copy(zeros_hbm.at[pl.ds(row0, RPT), :],
                    acc_sh.at[pl.ds(row0, RPT), :])
    plsc.subcore_barrier()

    def load_and_gather(i, b):
        off = base + i * CHUNK
        pltpu.sync_copy(src_hbm.at[pl.ds(off, CHUNK)], src_i[b])
        pltpu.sync_copy(dst_hbm.at[pl.ds(off, CHUNK)], dst_i[b])
        pltpu.async_copy(x_hbm.at[src_i[b]], rows[b], gsems[b])

    for b in range(NBUF):
        load_and_gather(b, b)

    def ring_body(p, carry):
        i0 = p * NBUF
        for b in range(NBUF):
            i = i0 + b
            pltpu.make_async_copy(x_hbm.at[src_i[b]], rows[b],
                                  gsems[b]).wait()
            pltpu.sync_copy(rows[b], acc_sh.at[dst_i[b]], add=True)
            nxt = i + NBUF

            @pl.when(nxt < CPT)
            def _():
                load_and_gather(nxt, b)

        return carry

    lax.fori_loop(0, CPT // NBUF, ring_body, 0)
    plsc.subcore_barrier()

    pltpu.sync_copy(acc_sh.at[pl.ds(row0, RPT), :],
                    out_hbm.at[c, pl.ds(row0, RPT), :])


BM = 1264


def _merge_body(p_ref, w_ref, b_ref, o_ref):
    acc = p_ref[0] + p_ref[1]
    o_ref[...] = lax.dot_general(
        acc, w_ref[...], (((1,), (1,)), ((), ())),
        preferred_element_type=jnp.float32) + b_ref[...]


def _merge(partial, w, b2d):
    return pl.pallas_call(
        _merge_body,
        grid=(NPAD // BM,),
        in_specs=[
            pl.BlockSpec((NC, BM, D), lambda i: (0, i, 0)),
            pl.BlockSpec((D, D), lambda i: (0, 0)),
            pl.BlockSpec((1, D), lambda i: (0, 0)),
        ],
        out_specs=pl.BlockSpec((BM, D), lambda i: (i, 0)),
        out_shape=jax.ShapeDtypeStruct((NPAD, D), jnp.float32),
    )(partial, w, b2d)


def kernel(x, edge_index, W, b):
    src = edge_index[0].astype(jnp.int32)
    dst = edge_index[1].astype(jnp.int32)
    pad = EPAD - E
    src = jnp.concatenate([src, jnp.zeros((pad,), jnp.int32)])
    dst = jnp.concatenate([dst, jnp.full((pad,), N, jnp.int32)])
    zeros = jnp.zeros((NPAD, D), jnp.float32)
    partial = _sc_aggregate(x, src, dst, zeros)
    out = _merge(partial, W, b.reshape(1, D))
    return out[:N]


# speedup vs baseline: 2.2550x; 2.2550x over previous
import functools

import jax
import jax.numpy as jnp
from jax import lax
from jax.experimental import pallas as pl
from jax.experimental.pallas import tpu as pltpu
from jax.experimental.pallas import tpu_sc as plsc

N = 10000
D = 128
E = 320000

NC = 2
NS = 16
NW = NC * NS

CHUNK = 128
NBUF = 2
NCHUNKS = E // CHUNK
CPT_LO = NCHUNKS // NW
REM = NCHUNKS - CPT_LO * NW

NPAD = 10112
RPT = NPAD // NS

_MESH = plsc.VectorSubcoreMesh(core_axis_name="c", subcore_axis_name="s")


@functools.partial(
    pl.kernel,
    out_type=jax.ShapeDtypeStruct((NC, NPAD, D), jnp.float32),
    mesh=_MESH,
    scratch_types=[
        [pltpu.VMEM((CHUNK,), jnp.int32) for _ in range(NBUF)],
        [pltpu.VMEM((CHUNK,), jnp.int32) for _ in range(NBUF)],
        [pltpu.VMEM((CHUNK, D), jnp.float32) for _ in range(NBUF)],
        pltpu.VMEM_SHARED((NPAD, D), jnp.float32),
        [pltpu.SemaphoreType.DMA for _ in range(NBUF)],
    ],
)
def _sc_aggregate(x_hbm, src_hbm, dst_hbm, zeros_hbm, out_hbm,
                  src_i, dst_i, rows, acc_sh, gsems):
    c = lax.axis_index("c")
    s = lax.axis_index("s")
    row0 = s * RPT
    wid = c * NS + s
    nch = CPT_LO + jnp.where(wid < REM, 1, 0)
    base = (wid * CPT_LO + jnp.minimum(wid, REM)) * CHUNK

    pltpu.sync_copy(zeros_hbm.at[pl.ds(row0, RPT), :],
                    acc_sh.at[pl.ds(row0, RPT), :])
    plsc.subcore_barrier()

    def load_and_gather(i, b):
        off = base + i * CHUNK
        pltpu.sync_copy(src_hbm.at[pl.ds(off, CHUNK)], src_i[b])
        pltpu.sync_copy(dst_hbm.at[pl.ds(off, CHUNK)], dst_i[b])
        pltpu.async_copy(x_hbm.at[src_i[b]], rows[b], gsems[b])

    def wait_and_scatter(b):
        pltpu.make_async_copy(x_hbm.at[src_i[b]], rows[b], gsems[b]).wait()
        pltpu.sync_copy(rows[b], acc_sh.at[dst_i[b]], add=True)

    for b in range(NBUF):
        load_and_gather(b, b)

    def ring_body(p, carry):
        i0 = p * NBUF
        for b in range(NBUF):
            i = i0 + b
            wait_and_scatter(b)
            nxt = i + NBUF

            @pl.when(nxt < nch)
            def _():
                load_and_gather(nxt, b)

        return carry

    lax.fori_loop(0, CPT_LO // NBUF, ring_body, 0)

    @pl.when(nch > CPT_LO)
    def _():
        wait_and_scatter(CPT_LO % NBUF)

    plsc.subcore_barrier()

    pltpu.sync_copy(acc_sh.at[pl.ds(row0, RPT), :],
                    out_hbm.at[c, pl.ds(row0, RPT), :])


BM = 1264


def _merge_body(p_ref, w_ref, b_ref, o_ref):
    acc = p_ref[0] + p_ref[1]
    o_ref[...] = lax.dot_general(
        acc, w_ref[...], (((1,), (1,)), ((), ())),
        preferred_element_type=jnp.float32) + b_ref[...]


def _merge(partial, w, b2d):
    return pl.pallas_call(
        _merge_body,
        grid=(NPAD // BM,),
        in_specs=[
            pl.BlockSpec((NC, BM, D), lambda i: (0, i, 0)),
            pl.BlockSpec((D, D), lambda i: (0, 0)),
            pl.BlockSpec((1, D), lambda i: (0, 0)),
        ],
        out_specs=pl.BlockSpec((BM, D), lambda i: (i, 0)),
        out_shape=jax.ShapeDtypeStruct((NPAD, D), jnp.float32),
    )(partial, w, b2d)


def kernel(x, edge_index, W, b):
    src = edge_index[0].astype(jnp.int32)
    dst = edge_index[1].astype(jnp.int32)
    zeros = jnp.zeros((NPAD, D), jnp.float32)
    partial = _sc_aggregate(x, src, dst, zeros)
    out = _merge(partial, W, b.reshape(1, D))
    return out[:N]


# speedup vs baseline: 2.2579x; 1.0013x over previous
import functools

import jax
import jax.numpy as jnp
from jax import lax
from jax.experimental import pallas as pl
from jax.experimental.pallas import tpu as pltpu
from jax.experimental.pallas import tpu_sc as plsc

N = 10000
D = 128
E = 320000

NC = 2
NS = 16
NW = NC * NS

CHUNK = 128
NBUF = 2
NCHUNKS = E // CHUNK
CPT_LO = NCHUNKS // NW
REM = NCHUNKS - CPT_LO * NW

NPAD = 10112
RPT = NPAD // NS

_MESH = plsc.VectorSubcoreMesh(core_axis_name="c", subcore_axis_name="s")


@functools.partial(
    pl.kernel,
    out_type=jax.ShapeDtypeStruct((NC, NPAD, D), jnp.float32),
    mesh=_MESH,
    scratch_types=[
        [pltpu.VMEM((CHUNK,), jnp.int32) for _ in range(NBUF)],
        [pltpu.VMEM((CHUNK,), jnp.int32) for _ in range(NBUF)],
        [pltpu.VMEM((CHUNK, D), jnp.float32) for _ in range(NBUF)],
        pltpu.VMEM_SHARED((NPAD, D), jnp.float32),
        [pltpu.SemaphoreType.DMA for _ in range(NBUF)],
    ],
)
def _sc_aggregate(x_hbm, src_hbm, dst_hbm, zeros_hbm, out_hbm,
                  src_i, dst_i, rows, acc_sh, gsems):
    c = lax.axis_index("c")
    s = lax.axis_index("s")
    row0 = s * RPT
    wid = c * NS + s
    nch = CPT_LO + jnp.where(wid < REM, 1, 0)
    base = (wid * CPT_LO + jnp.minimum(wid, REM)) * CHUNK

    pltpu.sync_copy(zeros_hbm.at[pl.ds(row0, RPT), :],
                    acc_sh.at[pl.ds(row0, RPT), :])
    plsc.subcore_barrier()

    def load_and_gather(i, b):
        off = base + i * CHUNK
        pltpu.sync_copy(src_hbm.at[pl.ds(off, CHUNK)], src_i[b])
        pltpu.sync_copy(dst_hbm.at[pl.ds(off, CHUNK)], dst_i[b])
        pltpu.async_copy(x_hbm.at[src_i[b]], rows[b], gsems[b])

    def wait_and_scatter(b):
        pltpu.make_async_copy(x_hbm.at[src_i[b]], rows[b], gsems[b]).wait()
        pltpu.sync_copy(rows[b], acc_sh.at[dst_i[b]], add=True)

    for b in range(NBUF):
        load_and_gather(b, b)

    def ring_body(p, carry):
        i0 = p * NBUF
        for b in range(NBUF):
            i = i0 + b
            wait_and_scatter(b)
            nxt = i + NBUF

            @pl.when(nxt < nch)
            def _():
                load_and_gather(nxt, b)

        return carry

    lax.fori_loop(0, CPT_LO // NBUF, ring_body, 0)

    for i in range(CPT_LO - CPT_LO % NBUF, CPT_LO):
        wait_and_scatter(i % NBUF)

    @pl.when(nch > CPT_LO)
    def _():
        wait_and_scatter(CPT_LO % NBUF)

    plsc.subcore_barrier()

    pltpu.sync_copy(acc_sh.at[pl.ds(row0, RPT), :],
                    out_hbm.at[c, pl.ds(row0, RPT), :])


BM = 1264


def _merge_body(p_ref, w_ref, b_ref, o_ref):
    acc = p_ref[0] + p_ref[1]
    o_ref[...] = lax.dot_general(
        acc, w_ref[...], (((1,), (1,)), ((), ())),
        preferred_element_type=jnp.float32) + b_ref[...]


def _merge(partial, w, b2d):
    return pl.pallas_call(
        _merge_body,
        grid=(NPAD // BM,),
        in_specs=[
            pl.BlockSpec((NC, BM, D), lambda i: (0, i, 0)),
            pl.BlockSpec((D, D), lambda i: (0, 0)),
            pl.BlockSpec((1, D), lambda i: (0, 0)),
        ],
        out_specs=pl.BlockSpec((BM, D), lambda i: (i, 0)),
        out_shape=jax.ShapeDtypeStruct((NPAD, D), jnp.float32),
    )(partial, w, b2d)


def kernel(x, edge_index, W, b):
    src = edge_index[0].astype(jnp.int32)
    dst = edge_index[1].astype(jnp.int32)
    zeros = jnp.zeros((NPAD, D), jnp.float32)
    partial = _sc_aggregate(x, src, dst, zeros)
    out = _merge(partial, W, b.reshape(1, D))
    return out[:N]


# speedup vs baseline: 2.4171x; 1.0705x over previous
import functools

import jax
import jax.numpy as jnp
from jax import lax
from jax.experimental import pallas as pl
from jax.experimental.pallas import tpu as pltpu
from jax.experimental.pallas import tpu_sc as plsc

N = 10000
D = 128
E = 320000

NC = 2
NS = 16
NW = NC * NS

CHUNK = 128
NBUF = 2
NCHUNKS = E // CHUNK
CPT_LO = NCHUNKS // NW
REM = NCHUNKS - CPT_LO * NW

NPAD = 10112
RPT = NPAD // NS

_MESH = plsc.VectorSubcoreMesh(core_axis_name="c", subcore_axis_name="s")


@functools.partial(
    pl.kernel,
    out_type=jax.ShapeDtypeStruct((NC, NPAD, D), jnp.float32),
    mesh=_MESH,
    scratch_types=[
        [pltpu.VMEM((CHUNK,), jnp.int32) for _ in range(NBUF)],
        [pltpu.VMEM((CHUNK,), jnp.int32) for _ in range(NBUF)],
        [pltpu.VMEM((CHUNK, D), jnp.float32) for _ in range(NBUF)],
        pltpu.VMEM_SHARED((NPAD, D), jnp.float32),
        [pltpu.SemaphoreType.DMA for _ in range(NBUF)],
    ],
)
def _sc_aggregate(x_hbm, edges_hbm, zeros_hbm, out_hbm,
                  src_i, dst_i, rows, acc_sh, gsems):
    c = lax.axis_index("c")
    s = lax.axis_index("s")
    row0 = s * RPT
    wid = c * NS + s
    nch = CPT_LO + jnp.where(wid < REM, 1, 0)
    base = (wid * CPT_LO + jnp.minimum(wid, REM)) * CHUNK

    pltpu.sync_copy(zeros_hbm.at[pl.ds(row0, RPT), :],
                    acc_sh.at[pl.ds(row0, RPT), :])
    plsc.subcore_barrier()

    def load_and_gather(i, b):
        off = base + i * CHUNK
        pltpu.sync_copy(edges_hbm.at[pl.ds(off, CHUNK)], src_i[b])
        pltpu.sync_copy(edges_hbm.at[pl.ds(E + off, CHUNK)], dst_i[b])
        pltpu.async_copy(x_hbm.at[src_i[b]], rows[b], gsems[b])

    def wait_and_scatter(b):
        pltpu.make_async_copy(x_hbm.at[src_i[b]], rows[b], gsems[b]).wait()
        pltpu.sync_copy(rows[b], acc_sh.at[dst_i[b]], add=True)

    for b in range(NBUF):
        load_and_gather(b, b)

    def ring_body(p, carry):
        i0 = p * NBUF
        for b in range(NBUF):
            i = i0 + b
            wait_and_scatter(b)
            nxt = i + NBUF

            @pl.when(nxt < nch)
            def _():
                load_and_gather(nxt, b)

        return carry

    lax.fori_loop(0, CPT_LO // NBUF, ring_body, 0)

    for i in range(CPT_LO - CPT_LO % NBUF, CPT_LO):
        wait_and_scatter(i % NBUF)

    @pl.when(nch > CPT_LO)
    def _():
        wait_and_scatter(CPT_LO % NBUF)

    plsc.subcore_barrier()

    pltpu.sync_copy(acc_sh.at[pl.ds(row0, RPT), :],
                    out_hbm.at[c, pl.ds(row0, RPT), :])


BM = 1000


def _merge_body(p_ref, w_ref, b_ref, o_ref):
    acc = p_ref[0] + p_ref[1]
    o_ref[...] = lax.dot_general(
        acc, w_ref[...], (((1,), (1,)), ((), ())),
        preferred_element_type=jnp.float32) + b_ref[...]


def _merge(partial, w, b2d):
    return pl.pallas_call(
        _merge_body,
        grid=(N // BM,),
        in_specs=[
            pl.BlockSpec((NC, BM, D), lambda i: (0, i, 0)),
            pl.BlockSpec((D, D), lambda i: (0, 0)),
            pl.BlockSpec((1, D), lambda i: (0, 0)),
        ],
        out_specs=pl.BlockSpec((BM, D), lambda i: (i, 0)),
        out_shape=jax.ShapeDtypeStruct((N, D), jnp.float32),
    )(partial, w, b2d)


def kernel(x, edge_index, W, b):
    zeros = jnp.zeros((NPAD, D), jnp.float32)
    partial = _sc_aggregate(x, edge_index.astype(jnp.int32).reshape(2 * E), zeros)
    return _merge(partial, W, b.reshape(1, D))
